# baseline (speedup 1.0000x reference)
import functools

import jax
import jax.numpy as jnp
from jax import lax
from jax.experimental import pallas as pl
from jax.experimental.pallas import tpu as pltpu
from jax.experimental.pallas import tpu_sc as plsc

ROWS = 200
COLS = 2 * 4096
N = ROWS * COLS
J = int((1.0 - 0.1) * N)
NW = 32
CHUNK = N // NW
L = 16
NWIN = 2
WC = CHUNK // NWIN
RW = 3200
GRL = 16
GR = N // RW
RPW = WC // RW
C1, C2, C3 = 1024, 2048, 1024

_CP = pltpu.CompilerParams(needs_layout_passes=False)


def _mesh():
    return plsc.VectorSubcoreMesh(
        core_axis_name="c", subcore_axis_name="s", num_cores=2, num_subcores=16
    )


def _wid():
    return lax.axis_index("s") * 2 + lax.axis_index("c")


def _iota():
    return lax.broadcasted_iota(jnp.int32, (L,), 0)


def _ploop(n, body, unroll=8):
    plsc.parallel_loop(0, n, 1, unroll=unroll)(body)


def _zero(ref, nwords):
    z = jnp.zeros((L,), jnp.int32)
    _ploop(nwords // L, lambda k: ref.__setitem__(pl.ds(k * L, L), z))


def _start_load(x_hbm, loff_hbm, lbuf, idxbuf, data, sems):
    pltpu.sync_copy(loff_hbm, lbuf)
    idxbuf[...] = lbuf[...] + lax.broadcast(_wid() * GRL, (L,)) + _iota()
    return [
        pltpu.async_copy(
            x_hbm.at[idxbuf.at[pl.ds(w * RPW, RPW)]],
            data.at[pl.ds(w * RPW, RPW)],
            sems[w],
        )
        for w in range(NWIN)
    ]


def _hist_windows(data, lh, c, bucket_fn, cps):
    ones = jnp.ones((L,), jnp.int32)
    lane_base = _iota() * c

    for w in range(NWIN):
        cps[w].wait()
        for r in range(RPW):
            def body(k, r=w * RPW + r):
                x = data[r, pl.ds(k * L, L)]
                bits = lax.bitcast_convert_type(jnp.abs(x), jnp.int32)
                bucket, mask = bucket_fn(bits)
                plsc.addupdate_scatter(lh, [lane_base + bucket], ones, mask=mask)

            _ploop(RW // L, body)


def _lane_reduce(lh, red, c):
    def body(k):
        s = lh[pl.ds(k * L, L)]
        for l in range(1, L):
            s = s + lh[pl.ds(l * c + k * L, L)]
        red[pl.ds(k * L, L)] = s

    _ploop(c // L, body, unroll=2)


def _merge(hist_hbm, hr, acc, bs, c, base):
    for half in range(2):
        pltpu.sync_copy(hist_hbm.at[pl.ds(half * 16, 16)], hr)

        def body(k, half=half):
            s = hr[0, pl.ds(k * L, L)]
            for r in range(1, 16):
                s = s + hr[r, pl.ds(k * L, L)]
            if half == 0:
                acc[pl.ds(k * L, L)] = s
            else:
                s = acc[pl.ds(k * L, L)] + s
                acc[pl.ds(k * L, L)] = s
                bs[pl.ds(k * L, L)] = lax.broadcast(jnp.sum(s), (L,))

        _ploop(c // L, body, unroll=2)

    jv = jnp.full((L,), J, jnp.int32)
    iota = _iota()
    nb = c // L
    ng = nb // L

    def sel_group(g, carry):
        nblk, lcnt, run = carry
        idx = (lax.broadcast(g * L, (L,)) + iota) * L
        v = plsc.load_gather(bs, [idx])
        cs = plsc.cumsum(v) + lax.broadcast(run, (L,))
        m = (cs <= jv).astype(jnp.int32)
        nblk = nblk + jnp.sum(m)
        lcnt = lcnt + jnp.sum(v * m)
        run = run + jnp.sum(v)
        return nblk, lcnt, run

    zero = jnp.int32(0)
    nblk, lcnt, _ = lax.fori_loop(0, ng, sel_group, (zero, base, base))

    v = acc[pl.ds(nblk * L, L)]
    cs = plsc.cumsum(v) + lax.broadcast(lcnt, (L,))
    m = (cs <= jv).astype(jnp.int32)
    bsel = nblk * L + jnp.sum(m)
    lsel = lcnt + jnp.sum(v * m)
    return bsel, lsel


def _splat(s):
    return lax.broadcast(s, (L,))


@functools.cache
def _build():
    mesh = _mesh()
    dma4 = (pltpu.SemaphoreType.DMA,) * NWIN

    @functools.partial(
        pl.kernel,
        out_type=jax.ShapeDtypeStruct((NW, C1), jnp.int32),
        mesh=mesh,
        compiler_params=_CP,
        scratch_types=[
            pltpu.VMEM((L,), jnp.int32),
            pltpu.VMEM((L,), jnp.int32),
            pltpu.VMEM((GRL, RW), jnp.float32),
            pltpu.VMEM((L * C1,), jnp.int32),
            pltpu.VMEM((C1,), jnp.int32),
            dma4,
        ],
    )
    def _k1(x_hbm, loff_hbm, h1_hbm, lbuf, idxbuf, data, lh, red, sems):
        cps = _start_load(x_hbm, loff_hbm, lbuf, idxbuf, data, sems)
        _zero(lh, L * C1)
        _hist_windows(data, lh, C1, lambda bits: (bits >> 21, None), cps)
        _lane_reduce(lh, red, C1)
        pltpu.sync_copy(red, h1_hbm.at[_wid()])

    @functools.partial(
        pl.kernel,
        out_type=(
            jax.ShapeDtypeStruct((NW, C2), jnp.int32),
            jax.ShapeDtypeStruct((NW, 2, L), jnp.int32),
        ),
        mesh=mesh,
        compiler_params=_CP,
        scratch_types=[
            pltpu.VMEM((L,), jnp.int32),
            pltpu.VMEM((L,), jnp.int32),
            pltpu.VMEM((GRL, RW), jnp.float32),
            pltpu.VMEM((16, C1), jnp.int32),
            pltpu.VMEM((C1,), jnp.int32),
            pltpu.VMEM((C1,), jnp.int32),
            pltpu.VMEM((L * C2,), jnp.int32),
            pltpu.VMEM((C2,), jnp.int32),
            pltpu.VMEM((2, L), jnp.int32),
            dma4,
        ],
    )
    def _k2(x_hbm, loff_hbm, h1_hbm, h2_hbm, scal_hbm,
            lbuf, idxbuf, data, hr, acc, bs, lh, red, sb, sems):
        cps = _start_load(x_hbm, loff_hbm, lbuf, idxbuf, data, sems)
        b0, l0 = _merge(h1_hbm, hr, acc, bs, C1, jnp.int32(0))
        b0v = _splat(b0)
        _zero(lh, L * C2)

        def bucket_fn(bits):
            return (bits >> 10) & (C2 - 1), (bits >> 21) == b0v

        _hist_windows(data, lh, C2, bucket_fn, cps)
        _lane_reduce(lh, red, C2)
        pltpu.sync_copy(red, h2_hbm.at[_wid()])
        sb[0, :] = b0v
        sb[1, :] = _splat(l0)
        pltpu.sync_copy(sb, scal_hbm.at[_wid()])

    @functools.partial(
        pl.kernel,
        out_type=(
            jax.ShapeDtypeStruct((NW, C3), jnp.int32),
            jax.ShapeDtypeStruct((NW, 3, L), jnp.int32),
        ),
        mesh=mesh,
        compiler_params=_CP,
        scratch_types=[
            pltpu.VMEM((L,), jnp.int32),
            pltpu.VMEM((L,), jnp.int32),
            pltpu.VMEM((GRL, RW), jnp.float32),
            pltpu.VMEM((16, C2), jnp.int32),
            pltpu.VMEM((C2,), jnp.int32),
            pltpu.VMEM((C2,), jnp.int32),
            pltpu.VMEM((L * C3,), jnp.int32),
            pltpu.VMEM((C3,), jnp.int32),
            pltpu.VMEM((2, L), jnp.int32),
            pltpu.VMEM((3, L), jnp.int32),
            dma4,
        ],
    )
    def _k3(x_hbm, loff_hbm, h2_hbm, scal1_hbm, h3_hbm, scal_hbm,
            lbuf, idxbuf, data, hr, acc, bs, lh, red, si, sb, sems):
        cps = _start_load(x_hbm, loff_hbm, lbuf, idxbuf, data, sems)
        pltpu.sync_copy(scal1_hbm.at[0], si)
        b0v = si[0, :]
        l0 = jnp.max(si[1, :])
        b1, l1 = _merge(h2_hbm, hr, acc, bs, C2, l0)
        _zero(lh, L * C3)
        prefix = (b0v << 11) | _splat(b1)

        def bucket_fn(bits):
            return bits & (C3 - 1), (bits >> 10) == prefix

        _hist_windows(data, lh, C3, bucket_fn, cps)
        _lane_reduce(lh, red, C3)
        pltpu.sync_copy(red, h3_hbm.at[_wid()])
        sb[0, :] = b0v
        sb[1, :] = _splat(b1)
        sb[2, :] = _splat(l1)
        pltpu.sync_copy(sb, scal_hbm.at[_wid()])

    @functools.partial(
        pl.kernel,
        out_type=jax.ShapeDtypeStruct((N,), jnp.float32),
        mesh=mesh,
        compiler_params=_CP,
        scratch_types=[
            pltpu.VMEM((L,), jnp.int32),
            pltpu.VMEM((L,), jnp.int32),
            pltpu.VMEM((GRL, RW), jnp.float32),
            pltpu.VMEM((16, C3), jnp.int32),
            pltpu.VMEM((C3,), jnp.int32),
            pltpu.VMEM((C3,), jnp.int32),
            pltpu.VMEM((3, L), jnp.int32),
            dma4,
            dma4,
        ],
    )
    def _k4(x_hbm, loff_hbm, h3_hbm, scal_hbm, out_hbm,
            lbuf, idxbuf, data, hr, acc, bs, si, sems, osems):
        cps = _start_load(x_hbm, loff_hbm, lbuf, idxbuf, data, sems)
        pltpu.sync_copy(scal_hbm.at[0], si)
        b0v = si[0, :]
        b1v = si[1, :]
        l1 = jnp.max(si[2, :])
        b2, _ = _merge(h3_hbm, hr, acc, bs, C3, l1)
        t = (b0v << 21) | (b1v << 10) | _splat(b2)
        one = jnp.ones((L,), jnp.float32)
        fzero = jnp.zeros((L,), jnp.float32)
        base = _wid() * CHUNK

        ocps = []
        for w in range(NWIN):
            cps[w].wait()
            for r in range(w * RPW, (w + 1) * RPW):
                def body(k, r=r):
                    x = data[r, pl.ds(k * L, L)]
                    bits = lax.bitcast_convert_type(jnp.abs(x), jnp.int32)
                    data[r, pl.ds(k * L, L)] = jnp.where(bits >= t, one, fzero)

                _ploop(RW // L, body)
                ocps.append(
                    pltpu.async_copy(
                        data.at[r],
                        out_hbm.at[pl.ds(base + r * RW, RW)],
                        osems[r % NWIN],
                    )
                )
        for cp in ocps:
            cp.wait()

    return _k1, _k2, _k3, _k4


def kernel(adapters_mask, layer_idx):
    _k1, _k2, _k3, _k4 = _build()
    x = adapters_mask.reshape(adapters_mask.shape[0] * GR, RW)
    loff = jnp.full((L,), jnp.int32(layer_idx) * GR, jnp.int32)
    h1 = _k1(x, loff)
    h2, scal1 = _k2(x, loff, h1)
    h3, scal2 = _k3(x, loff, h2, scal1)
    out = _k4(x, loff, h3, scal2)
    return out.reshape(ROWS, COLS)


# speedup vs baseline: 3.0317x; 3.0317x over previous
import functools

import jax
import jax.numpy as jnp
from jax import lax
from jax.experimental import pallas as pl
from jax.experimental.pallas import tpu as pltpu
from jax.experimental.pallas import tpu_sc as plsc

ROWS = 200
COLS = 2 * 4096
N = ROWS * COLS
J = int((1.0 - 0.1) * N)
NW = 32
CHUNK = N // NW
L = 16
NWIN = 4
WC = CHUNK // NWIN
C1, C2, C3 = 1024, 2048, 1024

_CP = pltpu.CompilerParams(needs_layout_passes=False, disable_bounds_checks=True)


def _mesh():
    return plsc.VectorSubcoreMesh(
        core_axis_name="c", subcore_axis_name="s", num_cores=2, num_subcores=16
    )


def _wid():
    return lax.axis_index("s") * 2 + lax.axis_index("c")


def _iota():
    return lax.broadcasted_iota(jnp.int32, (L,), 0)


def _ploop(n, body, unroll=8):
    plsc.parallel_loop(0, n, 1, unroll=unroll)(body)


def _zero(ref, nwords):
    z = jnp.zeros((L,), jnp.int32)
    _ploop(nwords // L, lambda k: ref.__setitem__(pl.ds(k * L, L), z))


def _start_load(x_hbm, data, sems):
    base = _wid() * CHUNK
    return [
        pltpu.async_copy(
            x_hbm.at[pl.ds(base + w * WC, WC)], data.at[pl.ds(w * WC, WC)], sems[w]
        )
        for w in range(NWIN)
    ]


def _hist_windows(data, lh, c, bucket_fn, cps):
    ones = jnp.ones((L,), jnp.int32)
    lane_base = _iota() * c

    for w in range(NWIN):
        cps[w].wait()

        def body(k, w=w):
            x = data[pl.ds(w * WC + k * L, L)]
            bits = lax.bitcast_convert_type(jnp.abs(x), jnp.int32)
            bucket, mask = bucket_fn(bits)
            plsc.addupdate_scatter(lh, [lane_base + bucket], ones, mask=mask)

        _ploop(WC // L, body)


def _lane_reduce(lh, red, c):
    def body(k):
        s = lh[pl.ds(k * L, L)]
        for l in range(1, L):
            s = s + lh[pl.ds(l * c + k * L, L)]
        red[pl.ds(k * L, L)] = s

    _ploop(c // L, body, unroll=2)


def _merge(hist_hbm, hr, acc, bs, c, base):
    for half in range(2):
        pltpu.sync_copy(hist_hbm.at[pl.ds(half * 16, 16)], hr)

        def body(k, half=half):
            s = hr[0, pl.ds(k * L, L)]
            for r in range(1, 16):
                s = s + hr[r, pl.ds(k * L, L)]
            if half == 0:
                acc[pl.ds(k * L, L)] = s
            else:
                s = acc[pl.ds(k * L, L)] + s
                acc[pl.ds(k * L, L)] = s
                bs[pl.ds(k * L, L)] = lax.broadcast(jnp.sum(s), (L,))

        _ploop(c // L, body, unroll=2)

    jv = jnp.full((L,), J, jnp.int32)
    iota = _iota()
    nb = c // L
    ng = nb // L

    def sel_group(g, carry):
        nblk, lcnt, run = carry
        idx = (lax.broadcast(g * L, (L,)) + iota) * L
        v = plsc.load_gather(bs, [idx])
        cs = plsc.cumsum(v) + lax.broadcast(run, (L,))
        m = (cs <= jv).astype(jnp.int32)
        nblk = nblk + jnp.sum(m)
        lcnt = lcnt + jnp.sum(v * m)
        run = run + jnp.sum(v)
        return nblk, lcnt, run

    zero = jnp.int32(0)
    nblk, lcnt, _ = lax.fori_loop(0, ng, sel_group, (zero, base, base))

    v = acc[pl.ds(nblk * L, L)]
    cs = plsc.cumsum(v) + lax.broadcast(lcnt, (L,))
    m = (cs <= jv).astype(jnp.int32)
    bsel = nblk * L + jnp.sum(m)
    lsel = lcnt + jnp.sum(v * m)
    return bsel, lsel


def _splat(s):
    return lax.broadcast(s, (L,))


@functools.cache
def _build():
    mesh = _mesh()
    dma4 = (pltpu.SemaphoreType.DMA,) * NWIN

    @functools.partial(
        pl.kernel,
        out_type=jax.ShapeDtypeStruct((NW, C1), jnp.int32),
        mesh=mesh,
        compiler_params=_CP,
        scratch_types=[
            pltpu.VMEM((CHUNK,), jnp.float32),
            pltpu.VMEM((L * C1,), jnp.int32),
            pltpu.VMEM((C1,), jnp.int32),
            dma4,
        ],
    )
    def _k1(x_hbm, h1_hbm, data, lh, red, sems):
        cps = _start_load(x_hbm, data, sems)
        _zero(lh, L * C1)
        _hist_windows(data, lh, C1, lambda bits: (bits >> 21, None), cps)
        _lane_reduce(lh, red, C1)
        pltpu.sync_copy(red, h1_hbm.at[_wid()])

    @functools.partial(
        pl.kernel,
        out_type=(
            jax.ShapeDtypeStruct((NW, C2), jnp.int32),
            jax.ShapeDtypeStruct((NW, 2, L), jnp.int32),
        ),
        mesh=mesh,
        compiler_params=_CP,
        scratch_types=[
            pltpu.VMEM((CHUNK,), jnp.float32),
            pltpu.VMEM((16, C1), jnp.int32),
            pltpu.VMEM((C1,), jnp.int32),
            pltpu.VMEM((C1,), jnp.int32),
            pltpu.VMEM((L * C2,), jnp.int32),
            pltpu.VMEM((C2,), jnp.int32),
            pltpu.VMEM((2, L), jnp.int32),
            dma4,
        ],
    )
    def _k2(x_hbm, h1_hbm, h2_hbm, scal_hbm,
            data, hr, acc, bs, lh, red, sb, sems):
        cps = _start_load(x_hbm, data, sems)
        b0, l0 = _merge(h1_hbm, hr, acc, bs, C1, jnp.int32(0))
        b0v = _splat(b0)
        _zero(lh, L * C2)

        def bucket_fn(bits):
            return (bits >> 10) & (C2 - 1), (bits >> 21) == b0v

        _hist_windows(data, lh, C2, bucket_fn, cps)
        _lane_reduce(lh, red, C2)
        pltpu.sync_copy(red, h2_hbm.at[_wid()])
        sb[0, :] = b0v
        sb[1, :] = _splat(l0)
        pltpu.sync_copy(sb, scal_hbm.at[_wid()])

    @functools.partial(
        pl.kernel,
        out_type=(
            jax.ShapeDtypeStruct((NW, C3), jnp.int32),
            jax.ShapeDtypeStruct((NW, 3, L), jnp.int32),
        ),
        mesh=mesh,
        compiler_params=_CP,
        scratch_types=[
            pltpu.VMEM((CHUNK,), jnp.float32),
            pltpu.VMEM((16, C2), jnp.int32),
            pltpu.VMEM((C2,), jnp.int32),
            pltpu.VMEM((C2,), jnp.int32),
            pltpu.VMEM((L * C3,), jnp.int32),
            pltpu.VMEM((C3,), jnp.int32),
            pltpu.VMEM((2, L), jnp.int32),
            pltpu.VMEM((3, L), jnp.int32),
            dma4,
        ],
    )
    def _k3(x_hbm, h2_hbm, scal1_hbm, h3_hbm, scal_hbm,
            data, hr, acc, bs, lh, red, si, sb, sems):
        cps = _start_load(x_hbm, data, sems)
        pltpu.sync_copy(scal1_hbm.at[0], si)
        b0v = si[0, :]
        l0 = jnp.max(si[1, :])
        b1, l1 = _merge(h2_hbm, hr, acc, bs, C2, l0)
        _zero(lh, L * C3)
        prefix = (b0v << 11) | _splat(b1)

        def bucket_fn(bits):
            return bits & (C3 - 1), (bits >> 10) == prefix

        _hist_windows(data, lh, C3, bucket_fn, cps)
        _lane_reduce(lh, red, C3)
        pltpu.sync_copy(red, h3_hbm.at[_wid()])
        sb[0, :] = b0v
        sb[1, :] = _splat(b1)
        sb[2, :] = _splat(l1)
        pltpu.sync_copy(sb, scal_hbm.at[_wid()])

    @functools.partial(
        pl.kernel,
        out_type=jax.ShapeDtypeStruct((N,), jnp.float32),
        mesh=mesh,
        compiler_params=_CP,
        scratch_types=[
            pltpu.VMEM((CHUNK,), jnp.float32),
            pltpu.VMEM((16, C3), jnp.int32),
            pltpu.VMEM((C3,), jnp.int32),
            pltpu.VMEM((C3,), jnp.int32),
            pltpu.VMEM((3, L), jnp.int32),
            dma4,
            dma4,
        ],
    )
    def _k4(x_hbm, h3_hbm, scal_hbm, out_hbm,
            data, hr, acc, bs, si, sems, osems):
        cps = _start_load(x_hbm, data, sems)
        pltpu.sync_copy(scal_hbm.at[0], si)
        b0v = si[0, :]
        b1v = si[1, :]
        l1 = jnp.max(si[2, :])
        b2, _ = _merge(h3_hbm, hr, acc, bs, C3, l1)
        t = (b0v << 21) | (b1v << 10) | _splat(b2)
        one = jnp.ones((L,), jnp.float32)
        fzero = jnp.zeros((L,), jnp.float32)
        base = _wid() * CHUNK

        ocps = []
        for w in range(NWIN):
            cps[w].wait()

            def body(k, w=w):
                x = data[pl.ds(w * WC + k * L, L)]
                bits = lax.bitcast_convert_type(jnp.abs(x), jnp.int32)
                data[pl.ds(w * WC + k * L, L)] = jnp.where(bits >= t, one, fzero)

            _ploop(WC // L, body)
            ocps.append(
                pltpu.async_copy(
                    data.at[pl.ds(w * WC, WC)],
                    out_hbm.at[pl.ds(base + w * WC, WC)],
                    osems[w],
                )
            )
        for cp in ocps:
            cp.wait()

    return _k1, _k2, _k3, _k4


def kernel(adapters_mask, layer_idx):
    _k1, _k2, _k3, _k4 = _build()
    x = jax.lax.dynamic_index_in_dim(
        adapters_mask, layer_idx, axis=0, keepdims=False
    ).reshape(-1)
    h1 = _k1(x)
    h2, scal1 = _k2(x, h1)
    h3, scal2 = _k3(x, h2, scal1)
    out = _k4(x, h3, scal2)
    return out.reshape(ROWS, COLS)


# speedup vs baseline: 3.1156x; 1.0277x over previous
import functools

import jax
import jax.numpy as jnp
from jax import lax
from jax.experimental import pallas as pl
from jax.experimental.pallas import tpu as pltpu
from jax.experimental.pallas import tpu_sc as plsc

ROWS = 200
COLS = 2 * 4096
N = ROWS * COLS
J = int((1.0 - 0.1) * N)
NW = 32
CHUNK = N // NW
L = 16
NWIN = 4
WC = CHUNK // NWIN
HC = N // 16
WB = 12800
C1, C2, C3 = 1024, 2048, 1024

_CP = pltpu.CompilerParams(needs_layout_passes=False, disable_bounds_checks=True)


def _mesh():
    return plsc.VectorSubcoreMesh(
        core_axis_name="c", subcore_axis_name="s", num_cores=2, num_subcores=16
    )


def _wid():
    return lax.axis_index("s") * 2 + lax.axis_index("c")


def _iota():
    return lax.broadcasted_iota(jnp.int32, (L,), 0)


def _ploop(n, body, unroll=8):
    plsc.parallel_loop(0, n, 1, unroll=unroll)(body)


def _zero(ref, nwords):
    z = jnp.zeros((L,), jnp.int32)
    _ploop(nwords // L, lambda k: ref.__setitem__(pl.ds(k * L, L), z))


def _start_load(x_hbm, data, sems):
    base = _wid() * CHUNK
    return [
        pltpu.async_copy(
            x_hbm.at[pl.ds(base + w * WC, WC)], data.at[pl.ds(w * WC, WC)], sems[w]
        )
        for w in range(NWIN)
    ]


def _hist_windows(data, lh, c, bucket_fn, cps):
    ones = jnp.ones((L,), jnp.int32)
    lane_base = _iota() * c

    for w in range(NWIN):
        cps[w].wait()

        def body(k, w=w):
            x = data[pl.ds(w * WC + k * L, L)]
            bits = lax.bitcast_convert_type(jnp.abs(x), jnp.int32)
            bucket, mask = bucket_fn(bits)
            plsc.addupdate_scatter(lh, [lane_base + bucket], ones, mask=mask)

        _ploop(WC // L, body)


def _lane_reduce(lh, red, c):
    def body(k):
        s = lh[pl.ds(k * L, L)]
        for l in range(1, L):
            s = s + lh[pl.ds(l * c + k * L, L)]
        red[pl.ds(k * L, L)] = s

    _ploop(c // L, body, unroll=2)


def _accum(hr, acc, bs, c, final, first):
    def body(k):
        s = hr[0, pl.ds(k * L, L)]
        for r in range(1, 16):
            s = s + hr[r, pl.ds(k * L, L)]
        if not first:
            s = acc[pl.ds(k * L, L)] + s
        acc[pl.ds(k * L, L)] = s
        if final:
            bs[pl.ds(k * L, L)] = lax.broadcast(jnp.sum(s), (L,))

    _ploop(c // L, body, unroll=2)


def _select(acc, bs, c, base):
    jv = jnp.full((L,), J, jnp.int32)
    iota = _iota()
    nb = c // L
    ng = nb // L

    def sel_group(g, carry):
        nblk, lcnt, run = carry
        idx = (lax.broadcast(g * L, (L,)) + iota) * L
        v = plsc.load_gather(bs, [idx])
        cs = plsc.cumsum(v) + lax.broadcast(run, (L,))
        m = (cs <= jv).astype(jnp.int32)
        nblk = nblk + jnp.sum(m)
        lcnt = lcnt + jnp.sum(v * m)
        run = run + jnp.sum(v)
        return nblk, lcnt, run

    zero = jnp.int32(0)
    nblk, lcnt, _ = lax.fori_loop(0, ng, sel_group, (zero, base, base))

    v = acc[pl.ds(nblk * L, L)]
    cs = plsc.cumsum(v) + lax.broadcast(lcnt, (L,))
    m = (cs <= jv).astype(jnp.int32)
    bsel = nblk * L + jnp.sum(m)
    lsel = lcnt + jnp.sum(v * m)
    return bsel, lsel


def _merge(hist_hbm, hr, acc, bs, c, base):
    for half in range(2):
        pltpu.sync_copy(hist_hbm.at[pl.ds(half * 16, 16)], hr)
        _accum(hr, acc, bs, c, final=(half == 1), first=(half == 0))
    return _select(acc, bs, c, base)


def _splat(s):
    return lax.broadcast(s, (L,))


@functools.cache
def _build():
    mesh = _mesh()
    dma4 = (pltpu.SemaphoreType.DMA,) * NWIN

    @functools.partial(
        pl.kernel,
        out_type=jax.ShapeDtypeStruct((NW, C1), jnp.int32),
        mesh=mesh,
        compiler_params=_CP,
        scratch_types=[
            pltpu.VMEM((CHUNK,), jnp.float32),
            pltpu.VMEM((L * C1,), jnp.int32),
            pltpu.VMEM((C1,), jnp.int32),
            dma4,
        ],
    )
    def _k1(x_hbm, h1_hbm, data, lh, red, sems):
        cps = _start_load(x_hbm, data, sems)
        _zero(lh, L * C1)
        _hist_windows(data, lh, C1, lambda bits: (bits >> 21, None), cps)
        _lane_reduce(lh, red, C1)
        pltpu.sync_copy(red, h1_hbm.at[_wid()])

    @functools.partial(
        pl.kernel,
        out_type=(
            jax.ShapeDtypeStruct((NW, C2), jnp.int32),
            jax.ShapeDtypeStruct((NW, 2, L), jnp.int32),
        ),
        mesh=mesh,
        compiler_params=_CP,
        scratch_types=[
            pltpu.VMEM((CHUNK,), jnp.float32),
            pltpu.VMEM((16, C1), jnp.int32),
            pltpu.VMEM((C1,), jnp.int32),
            pltpu.VMEM((C1,), jnp.int32),
            pltpu.VMEM((L * C2,), jnp.int32),
            pltpu.VMEM((C2,), jnp.int32),
            pltpu.VMEM((2, L), jnp.int32),
            dma4,
        ],
    )
    def _k2(x_hbm, h1_hbm, h2_hbm, scal_hbm,
            data, hr, acc, bs, lh, red, sb, sems):
        cps = _start_load(x_hbm, data, sems)
        b0, l0 = _merge(h1_hbm, hr, acc, bs, C1, jnp.int32(0))
        b0v = _splat(b0)
        _zero(lh, L * C2)

        def bucket_fn(bits):
            return (bits >> 10) & (C2 - 1), (bits >> 21) == b0v

        _hist_windows(data, lh, C2, bucket_fn, cps)
        _lane_reduce(lh, red, C2)
        pltpu.sync_copy(red, h2_hbm.at[_wid()])
        sb[0, :] = b0v
        sb[1, :] = _splat(l0)
        pltpu.sync_copy(sb, scal_hbm.at[_wid()])

    @functools.partial(
        pl.kernel,
        out_type=jax.ShapeDtypeStruct((N,), jnp.float32),
        mesh=mesh,
        compiler_params=_CP,
        scratch_types=[
            pltpu.VMEM((4, WB), jnp.float32),
            pltpu.VMEM((16, C2), jnp.int32),
            pltpu.VMEM((C2,), jnp.int32),
            pltpu.VMEM((C2,), jnp.int32),
            pltpu.VMEM((L * C3,), jnp.int32),
            pltpu.VMEM((C3,), jnp.int32),
            pltpu.VMEM((16, C3), jnp.int32),
            pltpu.VMEM((2, L), jnp.int32),
            pltpu.VMEM_SHARED((16, C3), jnp.int32),
            (pltpu.SemaphoreType.DMA,) * 4,
            (pltpu.SemaphoreType.DMA,) * 4,
        ],
    )
    def _k34(x_hbm, h2_hbm, scal1_hbm, out_hbm,
             buf, hr, acc, bs, lh, red, hr3, si, shared, lsems, osems):
        s = lax.axis_index("s")
        hbase = s * HC
        nwh = HC // WB
        cps = [
            pltpu.async_copy(
                x_hbm.at[pl.ds(hbase + w * WB, WB)], buf.at[w], lsems[w]
            )
            for w in range(4)
        ]
        pltpu.sync_copy(scal1_hbm.at[0], si)
        b0v = si[0, :]
        l0 = jnp.max(si[1, :])
        b1, l1 = _merge(h2_hbm, hr, acc, bs, C2, l0)
        _zero(lh, L * C3)
        prefix = (b0v << 11) | _splat(b1)
        ones = jnp.ones((L,), jnp.int32)
        lane_base = _iota() * C3

        for w in range(nwh):
            cps[w].wait()

            def body(k, w=w):
                x = buf[w % 4, pl.ds(k * L, L)]
                bits = lax.bitcast_convert_type(jnp.abs(x), jnp.int32)
                plsc.addupdate_scatter(
                    lh, [lane_base + (bits & (C3 - 1))], ones,
                    mask=(bits >> 10) == prefix,
                )

            _ploop(WB // L, body)
            if w + 4 < nwh:
                cps.append(
                    pltpu.async_copy(
                        x_hbm.at[pl.ds(hbase + (w + 4) * WB, WB)],
                        buf.at[w % 4],
                        lsems[w % 4],
                    )
                )

        _lane_reduce(lh, red, C3)
        pltpu.sync_copy(red, shared.at[s])
        plsc.subcore_barrier()
        pltpu.sync_copy(shared, hr3)
        _accum(hr3, acc, bs, C3, final=True, first=True)
        b2, _ = _select(acc, bs, C3, l1)
        t = (b0v << 21) | (b1 << 10) | _splat(b2)
        one = jnp.ones((L,), jnp.float32)
        fzero = jnp.zeros((L,), jnp.float32)

        mbase = hbase + lax.axis_index("c") * (HC // 2)
        nwm = (HC // 2) // WB
        mcps = [
            pltpu.async_copy(
                x_hbm.at[pl.ds(mbase + w * WB, WB)], buf.at[w], lsems[w]
            )
            for w in range(nwm)
        ]
        ocps = []
        for w in range(nwm):
            mcps[w].wait()

            def body(k, w=w):
                x = buf[w, pl.ds(k * L, L)]
                bits = lax.bitcast_convert_type(jnp.abs(x), jnp.int32)
                buf[w, pl.ds(k * L, L)] = jnp.where(bits >= t, one, fzero)

            _ploop(WB // L, body)
            ocps.append(
                pltpu.async_copy(
                    buf.at[w],
                    out_hbm.at[pl.ds(mbase + w * WB, WB)],
                    osems[w],
                )
            )
        for cp in ocps:
            cp.wait()

    return _k1, _k2, _k34


def kernel(adapters_mask, layer_idx):
    _k1, _k2, _k34 = _build()
    x = jax.lax.dynamic_index_in_dim(
        adapters_mask, layer_idx, axis=0, keepdims=False
    ).reshape(-1)
    h1 = _k1(x)
    h2, scal1 = _k2(x, h1)
    out = _k34(x, h2, scal1)
    return out.reshape(ROWS, COLS)
